# initial kernel scaffold (unmeasured)
import jax
import jax.numpy as jnp
from jax import lax
from jax.experimental import pallas as pl
from jax.experimental.pallas import tpu as pltpu

T = 512
D = 1024
V_LOC = 8192
C = 8
RC = T // C


def kernel(x, W):
    x = x.astype(jnp.bfloat16)
    W = W.astype(jnp.bfloat16)

    def body(x_ref, w_ref, out_ref, send_buf, recv_buf, send_sems, recv_sems):
        my_x = lax.axis_index("x")
        my_y = lax.axis_index("y")
        peer = (my_x, 1 - my_y)

        barrier_sem = pltpu.get_barrier_semaphore()
        pl.semaphore_signal(
            barrier_sem, inc=1, device_id=peer,
            device_id_type=pl.DeviceIdType.MESH,
        )
        pl.semaphore_wait(barrier_sem, 1)

        rdmas = []
        for c in range(C):
            sl = pl.ds(c * RC, RC)
            logits = jnp.dot(
                x_ref[sl, :], w_ref[:, :], preferred_element_type=jnp.float32
            )
            send_buf[sl, :] = logits.astype(jnp.bfloat16)
            rdma = pltpu.make_async_remote_copy(
                src_ref=send_buf.at[sl],
                dst_ref=recv_buf.at[sl],
                send_sem=send_sems.at[c],
                recv_sem=recv_sems.at[c],
                device_id=peer,
                device_id_type=pl.DeviceIdType.MESH,
            )
            rdma.start()
            rdmas.append(rdma)

        for c in range(C):
            sl = pl.ds(c * RC, RC)
            rdmas[c].wait_recv()
            mine = send_buf[sl, :].astype(jnp.float32)
            other = recv_buf[sl, :].astype(jnp.float32)
            m = jnp.maximum(
                mine.max(axis=1, keepdims=True),
                other.max(axis=1, keepdims=True),
            )
            e_m = jnp.exp(mine - m)
            e_o = jnp.exp(other - m)
            r = 1.0 / (
                e_m.sum(axis=1, keepdims=True) + e_o.sum(axis=1, keepdims=True)
            )
            out_ref[sl, pl.ds(my_y * V_LOC, V_LOC)] = (e_m * r).astype(
                jnp.bfloat16
            )
            out_ref[sl, pl.ds((1 - my_y) * V_LOC, V_LOC)] = (e_o * r).astype(
                jnp.bfloat16
            )

        for rdma in rdmas:
            rdma.wait_send()

    return pl.pallas_call(
        body,
        out_shape=jax.ShapeDtypeStruct((T, 2 * V_LOC), jnp.bfloat16),
        in_specs=[
            pl.BlockSpec(memory_space=pltpu.VMEM),
            pl.BlockSpec(memory_space=pltpu.VMEM),
        ],
        out_specs=pl.BlockSpec(memory_space=pltpu.VMEM),
        scratch_shapes=[
            pltpu.VMEM((T, V_LOC), jnp.bfloat16),
            pltpu.VMEM((T, V_LOC), jnp.bfloat16),
            pltpu.SemaphoreType.DMA((C,)),
            pltpu.SemaphoreType.DMA((C,)),
        ],
        compiler_params=pltpu.CompilerParams(collective_id=0),
    )(x, W)


# baseline (device time: 129084 ns/iter reference)
import jax
import jax.numpy as jnp
from jax import lax
from jax.experimental import pallas as pl
from jax.experimental.pallas import tpu as pltpu

T = 512
D = 1024
V_LOC = 8192
C = 8
RC = T // C


def kernel(x, W):
    x = x.astype(jnp.bfloat16)
    W = W.astype(jnp.bfloat16)

    def body(x_ref, w_ref, out_ref, send_buf, send_sems, recv_sems):
        my_x = lax.axis_index("x")
        my_y = lax.axis_index("y")
        peer = (my_x, 1 - my_y)
        my_col = pl.ds(my_y * V_LOC, V_LOC)
        other_col = pl.ds((1 - my_y) * V_LOC, V_LOC)

        barrier_sem = pltpu.get_barrier_semaphore()
        pl.semaphore_signal(
            barrier_sem, inc=1, device_id=peer,
            device_id_type=pl.DeviceIdType.MESH,
        )
        pl.semaphore_wait(barrier_sem, 1)

        rdmas = []
        for c in range(C):
            sl = pl.ds(c * RC, RC)
            logits = jnp.dot(
                x_ref[sl, :], w_ref[:, :], preferred_element_type=jnp.float32
            )
            send_buf[sl, :] = logits.astype(jnp.bfloat16)
            rdma = pltpu.make_async_remote_copy(
                src_ref=send_buf.at[sl],
                dst_ref=out_ref.at[sl, my_col],
                send_sem=send_sems.at[c],
                recv_sem=recv_sems.at[c],
                device_id=peer,
                device_id_type=pl.DeviceIdType.MESH,
            )
            rdma.start()
            rdmas.append(rdma)

        for c in range(C):
            sl = pl.ds(c * RC, RC)
            rdmas[c].wait_recv()
            mine = send_buf[sl, :].astype(jnp.float32)
            other = out_ref[sl, other_col].astype(jnp.float32)
            m = jnp.maximum(
                mine.max(axis=1, keepdims=True),
                other.max(axis=1, keepdims=True),
            )
            e_m = jnp.exp(mine - m)
            e_o = jnp.exp(other - m)
            r = 1.0 / (
                e_m.sum(axis=1, keepdims=True) + e_o.sum(axis=1, keepdims=True)
            )
            out_ref[sl, my_col] = (e_m * r).astype(jnp.bfloat16)
            out_ref[sl, other_col] = (e_o * r).astype(jnp.bfloat16)

        for rdma in rdmas:
            rdma.wait_send()

    return pl.pallas_call(
        body,
        out_shape=jax.ShapeDtypeStruct((T, 2 * V_LOC), jnp.bfloat16),
        in_specs=[
            pl.BlockSpec(memory_space=pltpu.VMEM),
            pl.BlockSpec(memory_space=pltpu.VMEM),
        ],
        out_specs=pl.BlockSpec(memory_space=pltpu.VMEM),
        scratch_shapes=[
            pltpu.VMEM((T, V_LOC), jnp.bfloat16),
            pltpu.SemaphoreType.DMA((C,)),
            pltpu.SemaphoreType.DMA((C,)),
        ],
        compiler_params=pltpu.CompilerParams(collective_id=0),
    )(x, W)


# device time: 114382 ns/iter; 1.1285x vs baseline; 1.1285x over previous
import jax
import jax.numpy as jnp
from jax import lax
from jax.experimental import pallas as pl
from jax.experimental.pallas import tpu as pltpu

T = 512
D = 1024
V_LOC = 8192
KC = 16
CW = V_LOC // KC


def kernel(x, W):
    def body(x_ref, w_hbm, out_ref, x_bf, w_buf, my_stats, peer_stats,
             w_sems, send_sems, recv_sems):
        my_x = lax.axis_index("x")
        my_y = lax.axis_index("y")
        peer = (my_x, 1 - my_y)
        my_col0 = my_y * V_LOC
        other_col0 = (1 - my_y) * V_LOC

        barrier_sem = pltpu.get_barrier_semaphore()
        pl.semaphore_signal(
            barrier_sem, inc=1, device_id=peer,
            device_id_type=pl.DeviceIdType.MESH,
        )
        pl.semaphore_wait(barrier_sem, 1)

        x_bf[...] = x_ref[...].astype(jnp.bfloat16)

        def w_dma(k):
            return pltpu.make_async_copy(
                w_hbm.at[:, pl.ds(k * CW, CW)],
                w_buf.at[k % 2],
                w_sems.at[k % 2],
            )

        w_dma(0).start()
        rdmas = []
        for k in range(KC):
            if k + 1 < KC:
                w_dma(k + 1).start()
            w_dma(k).wait()
            wk = w_buf[k % 2].astype(jnp.bfloat16)
            logits = jnp.dot(
                x_bf[...], wk, preferred_element_type=jnp.float32
            )
            m = logits.max(axis=1, keepdims=True)
            e = jnp.exp(logits - m)
            my_stats[0, :, k : k + 1] = m
            my_stats[1, :, k : k + 1] = e.sum(axis=1, keepdims=True)
            col = pl.ds(my_col0 + k * CW, CW)
            out_ref[:, col] = e.astype(jnp.bfloat16)
            rdma = pltpu.make_async_remote_copy(
                src_ref=out_ref.at[:, col],
                dst_ref=out_ref.at[:, col],
                send_sem=send_sems.at[k],
                recv_sem=recv_sems.at[k],
                device_id=peer,
                device_id_type=pl.DeviceIdType.MESH,
            )
            rdma.start()
            rdmas.append(rdma)

        stat_rdma = pltpu.make_async_remote_copy(
            src_ref=my_stats,
            dst_ref=peer_stats,
            send_sem=send_sems.at[KC],
            recv_sem=recv_sems.at[KC],
            device_id=peer,
            device_id_type=pl.DeviceIdType.MESH,
        )
        stat_rdma.start()

        for r in rdmas:
            r.wait_recv()
        stat_rdma.wait_recv()
        for r in rdmas:
            r.wait_send()
        stat_rdma.wait_send()

        mm = my_stats[0]
        ms = my_stats[1]
        pm = peer_stats[0]
        ps = peer_stats[1]
        m_fin = jnp.maximum(
            mm.max(axis=1, keepdims=True), pm.max(axis=1, keepdims=True)
        )
        wm = jnp.exp(mm - m_fin)
        wp = jnp.exp(pm - m_fin)
        s_fin = (ms * wm).sum(axis=1, keepdims=True) + (ps * wp).sum(
            axis=1, keepdims=True
        )
        inv = 1.0 / s_fin
        fac_mine = wm * inv
        fac_peer = wp * inv
        for k in range(KC):
            colm = pl.ds(my_col0 + k * CW, CW)
            colp = pl.ds(other_col0 + k * CW, CW)
            out_ref[:, colm] = (
                out_ref[:, colm].astype(jnp.float32) * fac_mine[:, k : k + 1]
            ).astype(jnp.bfloat16)
            out_ref[:, colp] = (
                out_ref[:, colp].astype(jnp.float32) * fac_peer[:, k : k + 1]
            ).astype(jnp.bfloat16)

    return pl.pallas_call(
        body,
        out_shape=jax.ShapeDtypeStruct((T, 2 * V_LOC), jnp.bfloat16),
        in_specs=[
            pl.BlockSpec(memory_space=pltpu.VMEM),
            pl.BlockSpec(memory_space=pl.ANY),
        ],
        out_specs=pl.BlockSpec(memory_space=pltpu.VMEM),
        scratch_shapes=[
            pltpu.VMEM((T, D), jnp.bfloat16),
            pltpu.VMEM((2, D, CW), jnp.float32),
            pltpu.VMEM((2, T, KC), jnp.float32),
            pltpu.VMEM((2, T, KC), jnp.float32),
            pltpu.SemaphoreType.DMA((2,)),
            pltpu.SemaphoreType.DMA((KC + 1,)),
            pltpu.SemaphoreType.DMA((KC + 1,)),
        ],
        compiler_params=pltpu.CompilerParams(collective_id=0),
    )(x, W)


# device time: 92650 ns/iter; 1.3932x vs baseline; 1.2346x over previous
import jax
import jax.numpy as jnp
from jax import lax
from jax.experimental import pallas as pl
from jax.experimental.pallas import tpu as pltpu

T = 512
TH = T // 2
D = 1024
V_LOC = 8192
KC = 8
CW = V_LOC // KC


def kernel(x, W):
    def body(x_ref, w_hbm, out_ref, x_bf, w_buf,
             my_stats, xn_stats, yn_stats, diag_stats,
             w_sems, sx_sems, sy_sems, fx_sems, fy_sems,
             rx_sems, ry_sems, rd_sems, st_send, st_recv):
        my_x = lax.axis_index("x")
        my_y = lax.axis_index("y")
        xn = (1 - my_x, my_y)
        yn = (my_x, 1 - my_y)

        my_rows = pl.ds(my_x * TH, TH)
        other_rows = pl.ds((1 - my_x) * TH, TH)
        my_col0 = my_y * V_LOC
        other_col0 = (1 - my_y) * V_LOC

        def rcopy(src, dst, ssem, rsem, dev):
            return pltpu.make_async_remote_copy(
                src_ref=src, dst_ref=dst, send_sem=ssem, recv_sem=rsem,
                device_id=dev, device_id_type=pl.DeviceIdType.MESH,
            )

        barrier_sem = pltpu.get_barrier_semaphore()
        for nbr in (xn, yn):
            pl.semaphore_signal(
                barrier_sem, inc=1, device_id=nbr,
                device_id_type=pl.DeviceIdType.MESH,
            )
        pl.semaphore_wait(barrier_sem, 2)

        x_bf[...] = x_ref[my_rows, :].astype(jnp.bfloat16)

        def w_dma(k):
            return pltpu.make_async_copy(
                w_hbm.at[:, pl.ds(k * CW, CW)],
                w_buf.at[k % 2],
                w_sems.at[k % 2],
            )

        w_dma(0).start()
        sends = []
        for k in range(KC):
            if k + 1 < KC:
                w_dma(k + 1).start()
            w_dma(k).wait()
            wk = w_buf[k % 2].astype(jnp.bfloat16)
            logits = jnp.dot(
                x_bf[...], wk, preferred_element_type=jnp.float32
            )
            m = logits.max(axis=1, keepdims=True)
            e = jnp.exp(logits - m)
            my_stats[0, :, k : k + 1] = m
            my_stats[1, :, k : k + 1] = e.sum(axis=1, keepdims=True)
            col = pl.ds(my_col0 + k * CW, CW)
            out_ref[my_rows, col] = e.astype(jnp.bfloat16)
            blk = out_ref.at[my_rows, col]
            sx = rcopy(blk, blk, sx_sems.at[k], rx_sems.at[k], xn)
            sy = rcopy(blk, blk, sy_sems.at[k], ry_sems.at[k], yn)
            sx.start()
            sy.start()
            sends += [sx, sy]

        st_x = rcopy(my_stats, xn_stats, st_send.at[0], st_recv.at[0], xn)
        st_y = rcopy(my_stats, yn_stats, st_send.at[1], st_recv.at[1], yn)
        st_x.start()
        st_y.start()
        sends += [st_x, st_y]

        xrecvs = []
        yrecvs = []
        for k in range(KC):
            xcol = pl.ds(my_col0 + k * CW, CW)
            xblk = out_ref.at[other_rows, xcol]
            xr = rcopy(xblk, xblk, sx_sems.at[k], rx_sems.at[k], xn)
            ycol = pl.ds(other_col0 + k * CW, CW)
            yblk = out_ref.at[my_rows, ycol]
            yr = rcopy(yblk, yblk, sy_sems.at[k], ry_sems.at[k], yn)
            xrecvs.append((xr, xblk))
            yrecvs.append((yr, yblk))
            if k % 2 == 1:
                xr.wait_recv()
                fwd = rcopy(xblk, xblk, fy_sems.at[k], rd_sems.at[k], yn)
                fwd.start()
                sends.append(fwd)
            else:
                yr.wait_recv()
                fwd = rcopy(yblk, yblk, fx_sems.at[k], rd_sems.at[k], xn)
                fwd.start()
                sends.append(fwd)

        st_xr = rcopy(my_stats, xn_stats, st_send.at[0], st_recv.at[0], xn)
        st_xr.wait_recv()
        st_f = rcopy(xn_stats, diag_stats, st_send.at[2], st_recv.at[2], yn)
        st_f.start()
        sends.append(st_f)

        for k in range(KC):
            if k % 2 == 0:
                xrecvs[k][0].wait_recv()
            else:
                yrecvs[k][0].wait_recv()
            dcol = pl.ds(other_col0 + k * CW, CW)
            dblk = out_ref.at[other_rows, dcol]
            rcopy(dblk, dblk, fx_sems.at[k], rd_sems.at[k], xn).wait_recv()
        rcopy(my_stats, yn_stats, st_send.at[1], st_recv.at[1], yn).wait_recv()
        rcopy(my_stats, diag_stats, st_send.at[2], st_recv.at[2], yn).wait_recv()

        for s in sends:
            s.wait_send()

        def factors(a_stats, b_stats):
            am, asum = a_stats[0], a_stats[1]
            bm, bsum = b_stats[0], b_stats[1]
            m_fin = jnp.maximum(
                am.max(axis=1, keepdims=True), bm.max(axis=1, keepdims=True)
            )
            ea = jnp.exp(am - m_fin)
            eb = jnp.exp(bm - m_fin)
            s_fin = (asum * ea).sum(axis=1, keepdims=True) + (
                bsum * eb
            ).sum(axis=1, keepdims=True)
            inv = 1.0 / s_fin
            return ea * inv, eb * inv

        fac_my, fac_yn = factors(my_stats, yn_stats)
        fac_xn, fac_diag = factors(xn_stats, diag_stats)

        for k in range(KC):
            colm = pl.ds(my_col0 + k * CW, CW)
            colo = pl.ds(other_col0 + k * CW, CW)
            for rows, col, fac in (
                (my_rows, colm, fac_my),
                (my_rows, colo, fac_yn),
                (other_rows, colm, fac_xn),
                (other_rows, colo, fac_diag),
            ):
                out_ref[rows, col] = (
                    out_ref[rows, col].astype(jnp.float32)
                    * fac[:, k : k + 1]
                ).astype(jnp.bfloat16)

    stat_shape = pltpu.VMEM((2, TH, KC), jnp.float32)
    return pl.pallas_call(
        body,
        out_shape=jax.ShapeDtypeStruct((T, 2 * V_LOC), jnp.bfloat16),
        in_specs=[
            pl.BlockSpec(memory_space=pltpu.VMEM),
            pl.BlockSpec(memory_space=pl.ANY),
        ],
        out_specs=pl.BlockSpec(memory_space=pltpu.VMEM),
        scratch_shapes=[
            pltpu.VMEM((TH, D), jnp.bfloat16),
            pltpu.VMEM((2, D, CW), jnp.float32),
            stat_shape,
            stat_shape,
            stat_shape,
            stat_shape,
            pltpu.SemaphoreType.DMA((2,)),
            pltpu.SemaphoreType.DMA((KC,)),
            pltpu.SemaphoreType.DMA((KC,)),
            pltpu.SemaphoreType.DMA((KC,)),
            pltpu.SemaphoreType.DMA((KC,)),
            pltpu.SemaphoreType.DMA((KC,)),
            pltpu.SemaphoreType.DMA((KC,)),
            pltpu.SemaphoreType.DMA((KC,)),
            pltpu.SemaphoreType.DMA((3,)),
            pltpu.SemaphoreType.DMA((3,)),
        ],
        compiler_params=pltpu.CompilerParams(collective_id=0),
    )(x, W)
